# baseline (speedup 1.0000x reference)
import functools

import jax
import jax.numpy as jnp
from jax import lax
from jax.experimental import pallas as pl
from jax.experimental.pallas import tpu as pltpu
from jax.experimental.pallas import tpu_sc as plsc

NC = 2
NS = 16
NW = NC * NS
CH = 128


def _spmm_body(x_hbm, src_hbm, dst_hbm, w_hbm, zeros_hbm, out_hbm,
               src_r, dst_r, w_r, rows_g, rows_v, xsp_sh, acc_sh,
               esem, gsem, ssem,
               *, n_chunks, n_real, d, dc, rows_per_tile, nbuf):
    cid = lax.axis_index("c")
    sid = lax.axis_index("s")
    dcb = dc // 32
    c0 = cid * dc
    c0b = cid * dcb

    r0 = sid * rows_per_tile
    pltpu.sync_copy(zeros_hbm.at[pl.ds(r0, rows_per_tile)],
                    acc_sh.at[pl.ds(r0, rows_per_tile)])
    last_rows = n_real - (NS - 1) * rows_per_tile

    @pl.when(sid < NS - 1)
    def _stage_full():
        pltpu.sync_copy(
            x_hbm.at[pl.ds(r0, rows_per_tile), pl.ds(c0b, dcb), :, :],
            xsp_sh.at[pl.ds(r0, rows_per_tile)])

    @pl.when(sid == NS - 1)
    def _stage_last():
        pltpu.sync_copy(
            x_hbm.at[pl.ds((NS - 1) * rows_per_tile, last_rows),
                     pl.ds(c0b, dcb), :, :],
            xsp_sh.at[pl.ds((NS - 1) * rows_per_tile, last_rows)])

    plsc.subcore_barrier()

    def fetch_edges(ch, slot):
        pltpu.async_copy(src_hbm.at[sid, ch], src_r.at[slot], esem.at[slot])
        pltpu.async_copy(dst_hbm.at[sid, ch], dst_r.at[slot], esem.at[slot])
        pltpu.async_copy(w_hbm.at[sid, ch], w_r.at[slot], esem.at[slot])

    def wait_edges(slot):
        pltpu.make_async_copy(src_hbm.at[sid, 0], src_r.at[slot],
                              esem.at[slot]).wait()
        pltpu.make_async_copy(dst_hbm.at[sid, 0], dst_r.at[slot],
                              esem.at[slot]).wait()
        pltpu.make_async_copy(w_hbm.at[sid, 0], w_r.at[slot],
                              esem.at[slot]).wait()

    def drain_scatter(b, slot):
        pltpu.make_async_copy(rows_v.at[b], acc_sh.at[dst_r.at[slot]],
                              ssem.at[b]).wait()

    def scale(gbuf, fbuf, slot):
        def group_body(g, c):
            wv16 = w_r[slot, pl.ds(g * 16, 16)]
            for k in range(16):
                wsplat = jnp.full((16,), wv16[k], jnp.float32)
                e = g * 16 + k
                for j in range(dc // 32):
                    cf = gbuf[e, j].astype(jnp.float32)
                    fbuf[e, pl.ds(j * 32, 16)] = cf[0] * wsplat
                    fbuf[e, pl.ds(j * 32 + 16, 16)] = cf[1] * wsplat
            return c

        lax.fori_loop(0, CH // 16, group_body, 0)

    n_groups = n_chunks // nbuf
    n_pairs = n_groups // 2

    def do_group(g_idx, par, is_first, is_last, p2):
        cur = [par * nbuf + b for b in range(nbuf)]
        nxt = [(1 - par) * nbuf + b for b in range(nbuf)]
        for b in range(nbuf):
            if is_first is not None:
                @pl.when(p2 > 0)
                def _d(b=b, s=cur[b]):
                    drain_scatter(b, s)
            else:
                drain_scatter(b, cur[b])
            if is_last is not None:
                @pl.when(p2 < n_pairs - 1)
                def _f(b=b, s=nxt[b]):
                    fetch_edges((g_idx + 1) * nbuf + b, s)
            else:
                fetch_edges((g_idx + 1) * nbuf + b, nxt[b])
            wait_edges(cur[b])
            pltpu.async_copy(xsp_sh.at[src_r.at[cur[b]]], rows_g.at[b],
                             gsem.at[b])
        for b in range(nbuf):
            pltpu.make_async_copy(xsp_sh.at[src_r.at[cur[b]]], rows_g.at[b],
                                  gsem.at[b]).wait()
            scale(rows_g.at[b], rows_v.at[b], cur[b])
            pltpu.async_copy(rows_v.at[b], acc_sh.at[dst_r.at[cur[b]]],
                             ssem.at[b], add=True)

    for b in range(nbuf):
        fetch_edges(b, b)

    def pair_body(p2, carry):
        do_group(2 * p2, 0, is_first=True, is_last=None, p2=p2)
        do_group(2 * p2 + 1, 1, is_first=None, is_last=True, p2=p2)
        return carry

    lax.fori_loop(0, n_pairs, pair_body, 0)
    for b in range(nbuf):
        drain_scatter(b, nbuf + b)
    plsc.subcore_barrier()
    pltpu.sync_copy(acc_sh.at[pl.ds(r0, rows_per_tile)],
                    out_hbm.at[pl.ds(r0, rows_per_tile), pl.ds(c0, dc)])


def _spmm_sc(x_packed, d, src3, dst3, w3, zeros):
    n_real = x_packed.shape[0]
    dc = d // NC
    n_pad = zeros.shape[0]
    n_chunks = src3.shape[1]
    rows_per_tile = n_pad // NS
    nbuf = 4
    mesh = plsc.VectorSubcoreMesh(core_axis_name="c", subcore_axis_name="s")
    body = functools.partial(_spmm_body, n_chunks=n_chunks, n_real=n_real,
                             d=d, dc=dc, rows_per_tile=rows_per_tile,
                             nbuf=nbuf)
    return pl.kernel(
        body,
        out_type=jax.ShapeDtypeStruct((n_pad, d), jnp.float32),
        mesh=mesh,
        compiler_params=pltpu.CompilerParams(use_tc_tiling_on_sc=False),
        scratch_types=[
            pltpu.VMEM((2 * nbuf, CH), jnp.int32),
            pltpu.VMEM((2 * nbuf, CH), jnp.int32),
            pltpu.VMEM((2 * nbuf, CH), jnp.float32),
            pltpu.VMEM((nbuf, CH, dc // 32, 2, 16), jnp.bfloat16),
            pltpu.VMEM((nbuf, CH, dc), jnp.float32),
            pltpu.VMEM_SHARED((n_pad, dc // 32, 2, 16), jnp.bfloat16),
            pltpu.VMEM_SHARED((n_pad, dc), jnp.float32),
            pltpu.SemaphoreType.DMA((2 * nbuf,)),
            pltpu.SemaphoreType.DMA((nbuf,)),
            pltpu.SemaphoreType.DMA((nbuf,)),
        ],
    )(x_packed, src3, dst3, w3, zeros)


def _layer1_body(x_ref, n_ref, w1_ref, b1_ref, w2_ref, y_ref, yb_ref):
    u = x_ref[...] + n_ref[...]
    h = jnp.dot(u, w1_ref[...], preferred_element_type=jnp.float32)
    h = jnp.maximum(h + b1_ref[...], 0.0)
    y = jnp.dot(h, w2_ref[...], preferred_element_type=jnp.float32)
    y_ref[...] = y
    yb_ref[...] = y.astype(jnp.bfloat16)


def _layer1_tc(x, n_arr, w1t, b1, w2t):
    n, d_in = x.shape
    d_hid = w1t.shape[1]
    d_out = w2t.shape[1]
    blk = 1000
    grid = n // blk
    return pl.pallas_call(
        _layer1_body,
        grid=(grid,),
        in_specs=[
            pl.BlockSpec((blk, d_in), lambda i: (i, 0)),
            pl.BlockSpec((blk, d_in), lambda i: (i, 0)),
            pl.BlockSpec((d_in, d_hid), lambda i: (0, 0)),
            pl.BlockSpec((1, d_hid), lambda i: (0, 0)),
            pl.BlockSpec((d_hid, d_out), lambda i: (0, 0)),
        ],
        out_specs=[
            pl.BlockSpec((blk, d_out), lambda i: (i, 0)),
            pl.BlockSpec((blk, d_out), lambda i: (i, 0)),
        ],
        out_shape=[
            jax.ShapeDtypeStruct((n, d_out), jnp.float32),
            jax.ShapeDtypeStruct((n, d_out), jnp.bfloat16),
        ],
    )(x, n_arr, w1t, b1, w2t)


def _final_body(y_ref, t_ref, b2_ref, o_ref):
    o_ref[...] = y_ref[...] + t_ref[...] + b2_ref[...]


def _final_tc(y, t_arr, b2):
    n, d_out = y.shape
    blk = 1000
    grid = n // blk
    return pl.pallas_call(
        _final_body,
        grid=(grid,),
        in_specs=[
            pl.BlockSpec((blk, d_out), lambda i: (i, 0)),
            pl.BlockSpec((blk, d_out), lambda i: (i, 0)),
            pl.BlockSpec((1, d_out), lambda i: (0, 0)),
        ],
        out_specs=pl.BlockSpec((blk, d_out), lambda i: (i, 0)),
        out_shape=jax.ShapeDtypeStruct((n, d_out), jnp.float32),
    )(y, t_arr, b2)


def kernel(feats, edge_index, edge_weight, W1, b1, W2, b2):
    n, d_in = feats.shape
    e = edge_weight.shape[0]
    src = edge_index[0].astype(jnp.int32)
    dst = edge_index[1].astype(jnp.int32)
    w = edge_weight.astype(jnp.float32)

    quantum = NS * CH
    n_chunks = -(-e // quantum)
    n_chunks = -(-n_chunks // 8) * 8
    e_pad = n_chunks * quantum
    pad = e_pad - e
    src = jnp.concatenate([src, jnp.zeros((pad,), jnp.int32)])
    dst = jnp.concatenate([dst, jnp.zeros((pad,), jnp.int32)])
    w = jnp.concatenate([w, jnp.zeros((pad,), jnp.float32)])
    src3 = src.reshape(NS, n_chunks, CH)
    dst3 = dst.reshape(NS, n_chunks, CH)
    w3 = w.reshape(NS, n_chunks, CH)

    n_pad = NS * (-(-n // (NS * 8))) * 8
    d_out = W2.shape[0]

    def pack_pairs(a):
        m, dd = a.shape
        return a.astype(jnp.bfloat16).reshape(m, dd // 32, 2, 16)

    w2t = W2.T
    zeros_hid = jnp.zeros((n_pad, d_in // NC), jnp.float32)
    n_arr = _spmm_sc(pack_pairs(feats), d_in, src3, dst3, w3,
                     zeros_hid)
    y, ybf = _layer1_tc(feats, n_arr, W1.T, b1.reshape(1, -1), w2t)
    zeros_out = jnp.zeros((n_pad, d_out // NC), jnp.float32)
    t_arr = _spmm_sc(ybf.reshape(n, d_out // 32, 2, 16), d_out, src3, dst3,
                     w3, zeros_out)
    return _final_tc(y, t_arr, b2.reshape(1, -1))


# speedup vs baseline: 2.1067x; 2.1067x over previous
import functools

import jax
import jax.numpy as jnp
from jax import lax
from jax.experimental import pallas as pl
from jax.experimental.pallas import tpu as pltpu
from jax.experimental.pallas import tpu_sc as plsc

NC = 2
NS = 16
NW = NC * NS
CH = 128


def _spmm_body(x_hbm, src_hbm, dst_hbm, w_hbm, zeros_hbm, out_hbm,
               src_r, dst_r, w_r, rows_v, xsp_sh, acc_sh,
               esem, gsem, ssem,
               *, n_chunks, n_real, d, dc, rows_per_tile, nbuf):
    cid = lax.axis_index("c")
    sid = lax.axis_index("s")
    c0 = cid * dc

    r0 = sid * rows_per_tile
    pltpu.sync_copy(zeros_hbm.at[pl.ds(r0, rows_per_tile)],
                    acc_sh.at[pl.ds(r0, rows_per_tile)])
    last_rows = n_real - (NS - 1) * rows_per_tile

    @pl.when(sid < NS - 1)
    def _stage_full():
        pltpu.sync_copy(
            x_hbm.at[pl.ds(r0, rows_per_tile), pl.ds(c0, dc)],
            xsp_sh.at[pl.ds(r0, rows_per_tile)])

    @pl.when(sid == NS - 1)
    def _stage_last():
        pltpu.sync_copy(
            x_hbm.at[pl.ds((NS - 1) * rows_per_tile, last_rows),
                     pl.ds(c0, dc)],
            xsp_sh.at[pl.ds((NS - 1) * rows_per_tile, last_rows)])

    plsc.subcore_barrier()

    def fetch_edges(ch, slot):
        pltpu.async_copy(src_hbm.at[sid, ch], src_r.at[slot], esem.at[slot])
        pltpu.async_copy(dst_hbm.at[sid, ch], dst_r.at[slot], esem.at[slot])
        pltpu.async_copy(w_hbm.at[sid, ch], w_r.at[slot], esem.at[slot])

    def wait_edges(slot):
        pltpu.make_async_copy(src_hbm.at[sid, 0], src_r.at[slot],
                              esem.at[slot]).wait()
        pltpu.make_async_copy(dst_hbm.at[sid, 0], dst_r.at[slot],
                              esem.at[slot]).wait()
        pltpu.make_async_copy(w_hbm.at[sid, 0], w_r.at[slot],
                              esem.at[slot]).wait()

    def drain_scatter(b, slot):
        pltpu.make_async_copy(rows_v.at[b], acc_sh.at[dst_r.at[slot]],
                              ssem.at[b]).wait()

    def scale(buf, slot):
        def group_body(g, c):
            wv16 = w_r[slot, pl.ds(g * 16, 16)]
            for k in range(16):
                wsplat = jnp.full((16,), wv16[k], jnp.float32)
                e = g * 16 + k
                for j in range(dc // 16):
                    sl = pl.ds(j * 16, 16)
                    buf[e, sl] = buf[e, sl] * wsplat
            return c

        lax.fori_loop(0, CH // 16, group_body, 0)

    n_groups = n_chunks // nbuf
    n_pairs = n_groups // 2

    def do_group(g_idx, par, is_first, is_last, p2):
        cur = [par * nbuf + b for b in range(nbuf)]
        nxt = [(1 - par) * nbuf + b for b in range(nbuf)]
        for b in range(nbuf):
            if is_first is not None:
                @pl.when(p2 > 0)
                def _d(b=b, s=cur[b]):
                    drain_scatter(b, s)
            else:
                drain_scatter(b, cur[b])
            if is_last is not None:
                @pl.when(p2 < n_pairs - 1)
                def _f(b=b, s=nxt[b]):
                    fetch_edges((g_idx + 1) * nbuf + b, s)
            else:
                fetch_edges((g_idx + 1) * nbuf + b, nxt[b])
            wait_edges(cur[b])
            pltpu.async_copy(xsp_sh.at[src_r.at[cur[b]]], rows_v.at[b],
                             gsem.at[b])
        for b in range(nbuf):
            pltpu.make_async_copy(xsp_sh.at[src_r.at[cur[b]]], rows_v.at[b],
                                  gsem.at[b]).wait()
            scale(rows_v.at[b], cur[b])
            pltpu.async_copy(rows_v.at[b], acc_sh.at[dst_r.at[cur[b]]],
                             ssem.at[b], add=True)

    for b in range(nbuf):
        fetch_edges(b, b)

    def pair_body(p2, carry):
        do_group(2 * p2, 0, is_first=True, is_last=None, p2=p2)
        do_group(2 * p2 + 1, 1, is_first=None, is_last=True, p2=p2)
        return carry

    lax.fori_loop(0, n_pairs, pair_body, 0)
    for b in range(nbuf):
        drain_scatter(b, nbuf + b)
    plsc.subcore_barrier()
    pltpu.sync_copy(acc_sh.at[pl.ds(r0, rows_per_tile)],
                    out_hbm.at[pl.ds(r0, rows_per_tile), pl.ds(c0, dc)])


def _spmm_sc(x_packed, d, src3, dst3, w3, zeros):
    n_real = x_packed.shape[0]
    dc = d // NC
    n_pad = zeros.shape[0]
    n_chunks = src3.shape[1]
    rows_per_tile = n_pad // NS
    nbuf = 5
    mesh = plsc.VectorSubcoreMesh(core_axis_name="c", subcore_axis_name="s")
    body = functools.partial(_spmm_body, n_chunks=n_chunks, n_real=n_real,
                             d=d, dc=dc, rows_per_tile=rows_per_tile,
                             nbuf=nbuf)
    return pl.kernel(
        body,
        out_type=jax.ShapeDtypeStruct((n_pad, d), jnp.float32),
        mesh=mesh,
        compiler_params=pltpu.CompilerParams(use_tc_tiling_on_sc=False),
        scratch_types=[
            pltpu.VMEM((2 * nbuf, CH), jnp.int32),
            pltpu.VMEM((2 * nbuf, CH), jnp.int32),
            pltpu.VMEM((2 * nbuf, CH), jnp.float32),
            pltpu.VMEM((nbuf, CH, dc), jnp.float32),
            pltpu.VMEM_SHARED((n_pad, dc), jnp.float32),
            pltpu.VMEM_SHARED((n_pad, dc), jnp.float32),
            pltpu.SemaphoreType.DMA((2 * nbuf,)),
            pltpu.SemaphoreType.DMA((nbuf,)),
            pltpu.SemaphoreType.DMA((nbuf,)),
        ],
    )(x_packed, src3, dst3, w3, zeros)


def _layer1_body(x_ref, n_ref, w1_ref, b1_ref, w2_ref, y_ref):
    u = x_ref[...] + n_ref[...]
    h = jnp.dot(u, w1_ref[...], preferred_element_type=jnp.float32)
    h = jnp.maximum(h + b1_ref[...], 0.0)
    y_ref[...] = jnp.dot(h, w2_ref[...], preferred_element_type=jnp.float32)


def _layer1_tc(x, n_arr, w1t, b1, w2t):
    n, d_in = x.shape
    d_hid = w1t.shape[1]
    d_out = w2t.shape[1]
    blk = 1000
    grid = n // blk
    return pl.pallas_call(
        _layer1_body,
        grid=(grid,),
        in_specs=[
            pl.BlockSpec((blk, d_in), lambda i: (i, 0)),
            pl.BlockSpec((blk, d_in), lambda i: (i, 0)),
            pl.BlockSpec((d_in, d_hid), lambda i: (0, 0)),
            pl.BlockSpec((1, d_hid), lambda i: (0, 0)),
            pl.BlockSpec((d_hid, d_out), lambda i: (0, 0)),
        ],
        out_specs=pl.BlockSpec((blk, d_out), lambda i: (i, 0)),
        out_shape=jax.ShapeDtypeStruct((n, d_out), jnp.float32),
    )(x, n_arr, w1t, b1, w2t)


def _final_body(y_ref, t_ref, b2_ref, o_ref):
    o_ref[...] = y_ref[...] + t_ref[...] + b2_ref[...]


def _final_tc(y, t_arr, b2):
    n, d_out = y.shape
    blk = 1000
    grid = n // blk
    return pl.pallas_call(
        _final_body,
        grid=(grid,),
        in_specs=[
            pl.BlockSpec((blk, d_out), lambda i: (i, 0)),
            pl.BlockSpec((blk, d_out), lambda i: (i, 0)),
            pl.BlockSpec((1, d_out), lambda i: (0, 0)),
        ],
        out_specs=pl.BlockSpec((blk, d_out), lambda i: (i, 0)),
        out_shape=jax.ShapeDtypeStruct((n, d_out), jnp.float32),
    )(y, t_arr, b2)


def kernel(feats, edge_index, edge_weight, W1, b1, W2, b2):
    n, d_in = feats.shape
    e = edge_weight.shape[0]
    src = edge_index[0].astype(jnp.int32)
    dst = edge_index[1].astype(jnp.int32)
    w = edge_weight.astype(jnp.float32)

    quantum = NS * CH
    n_chunks = -(-e // quantum)
    n_chunks = -(-n_chunks // 8) * 8
    e_pad = n_chunks * quantum
    pad = e_pad - e
    src = jnp.concatenate([src, jnp.zeros((pad,), jnp.int32)])
    dst = jnp.concatenate([dst, jnp.zeros((pad,), jnp.int32)])
    w = jnp.concatenate([w, jnp.zeros((pad,), jnp.float32)])
    src3 = src.reshape(NS, n_chunks, CH)
    dst3 = dst.reshape(NS, n_chunks, CH)
    w3 = w.reshape(NS, n_chunks, CH)

    n_pad = NS * (-(-n // (NS * 8))) * 8
    d_out = W2.shape[0]

    w2t = W2.T
    zeros_hid = jnp.zeros((n_pad, d_in // NC), jnp.float32)
    n_arr = _spmm_sc(feats, d_in, src3, dst3, w3, zeros_hid)
    y = _layer1_tc(feats, n_arr, W1.T, b1.reshape(1, -1), w2t)
    zeros_out = jnp.zeros((n_pad, d_out // NC), jnp.float32)
    t_arr = _spmm_sc(y, d_out, src3, dst3, w3, zeros_out)
    return _final_tc(y, t_arr, b2.reshape(1, -1))
